# baseline (speedup 1.0000x reference)
import jax
import jax.numpy as jnp
from jax.experimental import pallas as pl
from jax.experimental.pallas import tpu as pltpu

B = 16384
K = 64
BLKC = 1024
NBLK = B // BLKC


def _body(gu_hbm, gi_hbm, bu_ref, bi_ref, mu_ref, out_ref, gub, gib, sems):
    i = pl.program_id(0)

    def start_in(slot, idx):
        pltpu.make_async_copy(
            gu_hbm.at[:, pl.ds(idx * BLKC, BLKC)], gub.at[slot], sems.at[0, slot]
        ).start()
        pltpu.make_async_copy(
            gi_hbm.at[:, pl.ds(idx * BLKC, BLKC)], gib.at[slot], sems.at[1, slot]
        ).start()

    @pl.when(i == 0)
    def _():
        start_in(0, 0)

    @pl.when(i + 1 < NBLK)
    def _():
        start_in((i + 1) % 2, i + 1)

    slot = i % 2
    pltpu.make_async_copy(
        gu_hbm.at[:, pl.ds(i * BLKC, BLKC)], gub.at[slot], sems.at[0, slot]
    ).wait()
    pltpu.make_async_copy(
        gi_hbm.at[:, pl.ds(i * BLKC, BLKC)], gib.at[slot], sems.at[1, slot]
    ).wait()
    prod = gub[slot] * gib[slot]
    s = jnp.sum(prod, axis=0, keepdims=True)
    out_ref[...] = s + bu_ref[...] + bi_ref[...] + mu_ref[0, 0]


def kernel(gu, gi, bu, bi, Mu):
    gut = gu.T
    git = gi.T
    but = bu.T
    bit = bi.T
    grid = (NBLK,)
    out = pl.pallas_call(
        _body,
        grid=grid,
        in_specs=[
            pl.BlockSpec(memory_space=pl.ANY),
            pl.BlockSpec(memory_space=pl.ANY),
            pl.BlockSpec((1, BLKC), lambda i: (0, i)),
            pl.BlockSpec((1, BLKC), lambda i: (0, i)),
            pl.BlockSpec((1, 1), lambda i: (0, 0)),
        ],
        out_specs=pl.BlockSpec((1, BLKC), lambda i: (0, i)),
        out_shape=jax.ShapeDtypeStruct((1, B), jnp.float32),
        scratch_shapes=[
            pltpu.VMEM((2, K, BLKC), jnp.float32),
            pltpu.VMEM((2, K, BLKC), jnp.float32),
            pltpu.SemaphoreType.DMA((2, 2)),
        ],
        compiler_params=pltpu.CompilerParams(
            dimension_semantics=("arbitrary",),
        ),
    )(gut, git, but, bit, Mu)
    return out.reshape(B)


# speedup vs baseline: 1.2352x; 1.2352x over previous
import jax
import jax.numpy as jnp
from jax.experimental import pallas as pl
from jax.experimental.pallas import tpu as pltpu

B = 16384
K = 64
BLKC = 1024
NBLK = B // BLKC
NBUF = 4


def _body(gu_hbm, gi_hbm, bu_ref, bi_ref, mu_ref, out_ref, gub, gib, sems):
    i = pl.program_id(0)

    def start_in(slot, idx):
        pltpu.make_async_copy(
            gu_hbm.at[:, pl.ds(idx * BLKC, BLKC)], gub.at[slot], sems.at[0, slot]
        ).start()
        pltpu.make_async_copy(
            gi_hbm.at[:, pl.ds(idx * BLKC, BLKC)], gib.at[slot], sems.at[1, slot]
        ).start()

    @pl.when(i == 0)
    def _():
        for n in range(NBUF - 1):
            start_in(n, n)

    @pl.when(i + NBUF - 1 < NBLK)
    def _():
        start_in((i + NBUF - 1) % NBUF, i + NBUF - 1)

    slot = i % NBUF
    pltpu.make_async_copy(
        gu_hbm.at[:, pl.ds(i * BLKC, BLKC)], gub.at[slot], sems.at[0, slot]
    ).wait()
    pltpu.make_async_copy(
        gi_hbm.at[:, pl.ds(i * BLKC, BLKC)], gib.at[slot], sems.at[1, slot]
    ).wait()
    prod = gub[slot] * gib[slot]
    s = jnp.sum(prod, axis=0, keepdims=True)
    out_ref[...] = s + bu_ref[...] + bi_ref[...] + mu_ref[0, 0]


def kernel(gu, gi, bu, bi, Mu):
    gut = gu.T
    git = gi.T
    but = bu.T
    bit = bi.T
    grid = (NBLK,)
    out = pl.pallas_call(
        _body,
        grid=grid,
        in_specs=[
            pl.BlockSpec(memory_space=pl.ANY),
            pl.BlockSpec(memory_space=pl.ANY),
            pl.BlockSpec((1, BLKC), lambda i: (0, i)),
            pl.BlockSpec((1, BLKC), lambda i: (0, i)),
            pl.BlockSpec((1, 1), lambda i: (0, 0)),
        ],
        out_specs=pl.BlockSpec((1, BLKC), lambda i: (0, i)),
        out_shape=jax.ShapeDtypeStruct((1, B), jnp.float32),
        scratch_shapes=[
            pltpu.VMEM((NBUF, K, BLKC), jnp.float32),
            pltpu.VMEM((NBUF, K, BLKC), jnp.float32),
            pltpu.SemaphoreType.DMA((2, NBUF)),
        ],
        compiler_params=pltpu.CompilerParams(
            dimension_semantics=("arbitrary",),
        ),
    )(gut, git, but, bit, Mu)
    return out.reshape(B)


# speedup vs baseline: 2.0040x; 1.6224x over previous
import jax
import jax.numpy as jnp
from jax.experimental import pallas as pl
from jax.experimental.pallas import tpu as pltpu

B = 16384
K = 64
CHR = 8
NBLK = K // CHR
NBUF = 4


def _body(gu_hbm, gi_hbm, bu_ref, bi_ref, mu_ref, out_ref, gub, gib, sems):
    i = pl.program_id(0)

    def start_in(slot, idx):
        pltpu.make_async_copy(
            gu_hbm.at[pl.ds(idx * CHR, CHR), :], gub.at[slot], sems.at[0, slot]
        ).start()
        pltpu.make_async_copy(
            gi_hbm.at[pl.ds(idx * CHR, CHR), :], gib.at[slot], sems.at[1, slot]
        ).start()

    @pl.when(i == 0)
    def _():
        for n in range(NBUF - 1):
            start_in(n, n)

    @pl.when(i + NBUF - 1 < NBLK)
    def _():
        start_in((i + NBUF - 1) % NBUF, i + NBUF - 1)

    slot = i % NBUF
    pltpu.make_async_copy(
        gu_hbm.at[pl.ds(i * CHR, CHR), :], gub.at[slot], sems.at[0, slot]
    ).wait()
    pltpu.make_async_copy(
        gi_hbm.at[pl.ds(i * CHR, CHR), :], gib.at[slot], sems.at[1, slot]
    ).wait()
    partial = jnp.sum(gub[slot] * gib[slot], axis=0, keepdims=True)

    @pl.when(i == 0)
    def _():
        out_ref[...] = partial + bu_ref[...] + bi_ref[...] + mu_ref[0, 0]

    @pl.when(i > 0)
    def _():
        out_ref[...] = out_ref[...] + partial


def kernel(gu, gi, bu, bi, Mu):
    gut = gu.T
    git = gi.T
    but = bu.T
    bit = bi.T
    grid = (NBLK,)
    out = pl.pallas_call(
        _body,
        grid=grid,
        in_specs=[
            pl.BlockSpec(memory_space=pl.ANY),
            pl.BlockSpec(memory_space=pl.ANY),
            pl.BlockSpec((1, B), lambda i: (0, 0)),
            pl.BlockSpec((1, B), lambda i: (0, 0)),
            pl.BlockSpec((1, 1), lambda i: (0, 0)),
        ],
        out_specs=pl.BlockSpec((1, B), lambda i: (0, 0)),
        out_shape=jax.ShapeDtypeStruct((1, B), jnp.float32),
        scratch_shapes=[
            pltpu.VMEM((NBUF, CHR, B), jnp.float32),
            pltpu.VMEM((NBUF, CHR, B), jnp.float32),
            pltpu.SemaphoreType.DMA((2, NBUF)),
        ],
        compiler_params=pltpu.CompilerParams(
            dimension_semantics=("arbitrary",),
        ),
    )(gut, git, but, bit, Mu)
    return out.reshape(B)


# speedup vs baseline: 2.1332x; 1.0645x over previous
import jax
import jax.numpy as jnp
from jax.experimental import pallas as pl
from jax.experimental.pallas import tpu as pltpu

B = 16384
K = 64
CHR = 8
NBLK = K // CHR
NBUF = 8


def _body(gu_hbm, gi_hbm, bu_ref, bi_ref, mu_ref, out_ref, gub, gib, sems):
    i = pl.program_id(0)

    def start_in(slot, idx):
        pltpu.make_async_copy(
            gu_hbm.at[pl.ds(idx * CHR, CHR), :], gub.at[slot], sems.at[0, slot]
        ).start()
        pltpu.make_async_copy(
            gi_hbm.at[pl.ds(idx * CHR, CHR), :], gib.at[slot], sems.at[1, slot]
        ).start()

    @pl.when(i == 0)
    def _():
        for n in range(NBUF - 1):
            start_in(n, n)

    @pl.when(i + NBUF - 1 < NBLK)
    def _():
        start_in((i + NBUF - 1) % NBUF, i + NBUF - 1)

    slot = i % NBUF
    pltpu.make_async_copy(
        gu_hbm.at[pl.ds(i * CHR, CHR), :], gub.at[slot], sems.at[0, slot]
    ).wait()
    pltpu.make_async_copy(
        gi_hbm.at[pl.ds(i * CHR, CHR), :], gib.at[slot], sems.at[1, slot]
    ).wait()
    partial = jnp.sum(gub[slot] * gib[slot], axis=0, keepdims=True)

    @pl.when(i == 0)
    def _():
        out_ref[...] = partial + bu_ref[...] + bi_ref[...] + mu_ref[0, 0]

    @pl.when(i > 0)
    def _():
        out_ref[...] = out_ref[...] + partial


def kernel(gu, gi, bu, bi, Mu):
    gut = gu.T
    git = gi.T
    but = bu.T
    bit = bi.T
    grid = (NBLK,)
    out = pl.pallas_call(
        _body,
        grid=grid,
        in_specs=[
            pl.BlockSpec(memory_space=pl.ANY),
            pl.BlockSpec(memory_space=pl.ANY),
            pl.BlockSpec((1, B), lambda i: (0, 0)),
            pl.BlockSpec((1, B), lambda i: (0, 0)),
            pl.BlockSpec((1, 1), lambda i: (0, 0)),
        ],
        out_specs=pl.BlockSpec((1, B), lambda i: (0, 0)),
        out_shape=jax.ShapeDtypeStruct((1, B), jnp.float32),
        scratch_shapes=[
            pltpu.VMEM((NBUF, CHR, B), jnp.float32),
            pltpu.VMEM((NBUF, CHR, B), jnp.float32),
            pltpu.SemaphoreType.DMA((2, NBUF)),
        ],
        compiler_params=pltpu.CompilerParams(
            dimension_semantics=("arbitrary",),
        ),
    )(gut, git, but, bit, Mu)
    return out.reshape(B)
